# baseline (speedup 1.0000x reference)
import functools

import jax
import jax.numpy as jnp
import numpy as _np
from jax import lax
from jax.experimental import pallas as pl
from jax.experimental.pallas import tpu as pltpu
from jax.experimental.pallas import tpu_sc as plsc

N = 50000
DIN = 128
HID = 64
HEADS = 4
G = 64
NQ = 8
QW = HID // NQ
AB = 8

E_RAW = 800000
E_ALL = E_RAW + N
NSUB = 16
EPT = 53248
E_PAD = EPT * NSUB
BLK_A = 512
BLK_B = 256
NBLK_A = EPT // BLK_A
NBLK_B = EPT // BLK_B
N_PAD = 50048
ROWS_PT = N_PAD // NSUB
BN = 1000
NGRID = N // BN

_NEG = 0.2


def _tc_first_body(x_ref, win_ref, bin_ref, wp_ref, asel_ref, bsel_ref,
                   h_ref, hwp_ref, ab_ref, ms_ref, md_ref):
    i = pl.program_id(0)
    h = jnp.dot(x_ref[...], win_ref[...], preferred_element_type=jnp.float32)
    h = h + bin_ref[...]
    h_ref[...] = h
    hwp = jnp.dot(h, wp_ref[...], preferred_element_type=jnp.float32)
    hwp_ref[...] = hwp
    als = jnp.dot(hwp, asel_ref[...], preferred_element_type=jnp.float32)
    ald = jnp.dot(hwp, bsel_ref[...], preferred_element_type=jnp.float32)
    ab_ref[...] = jnp.concatenate([als, ald], axis=1)
    bs = jnp.max(als, axis=0)
    bd = jnp.max(ald, axis=0)

    @pl.when(i == 0)
    def _():
        ms_ref[...] = bs
        md_ref[...] = bd

    @pl.when(i != 0)
    def _():
        ms_ref[...] = jnp.maximum(ms_ref[...], bs)
        md_ref[...] = jnp.maximum(md_ref[...], bd)


def _tc_mid_body(acc_ref, hp_ref, bp_ref, wp_ref, asel_ref, bsel_ref,
                 h_ref, hwp_ref, ab_ref, ms_ref, md_ref):
    i = pl.program_id(0)
    t = acc_ref[...] * (1.0 / HEADS) + bp_ref[...]
    h = jnp.where(t > 0, t, jnp.exp(t) - 1.0) + hp_ref[...]
    h_ref[...] = h
    hwp = jnp.dot(h, wp_ref[...], preferred_element_type=jnp.float32)
    hwp_ref[...] = hwp
    als = jnp.dot(hwp, asel_ref[...], preferred_element_type=jnp.float32)
    ald = jnp.dot(hwp, bsel_ref[...], preferred_element_type=jnp.float32)
    ab_ref[...] = jnp.concatenate([als, ald], axis=1)
    bs = jnp.max(als, axis=0)
    bd = jnp.max(ald, axis=0)

    @pl.when(i == 0)
    def _():
        ms_ref[...] = bs
        md_ref[...] = bd

    @pl.when(i != 0)
    def _():
        ms_ref[...] = jnp.maximum(ms_ref[...], bs)
        md_ref[...] = jnp.maximum(md_ref[...], bd)


def _tc_pool_body(acc_ref, hp_ref, bp_ref, batch_ref, w1_ref, b1_ref,
                  w2_ref, b2_ref, out_ref, psum_ref, cnt_ref):
    i = pl.program_id(0)
    t = acc_ref[...] * (1.0 / HEADS) + bp_ref[...]
    h = jnp.where(t > 0, t, jnp.exp(t) - 1.0) + hp_ref[...]
    b = batch_ref[0, 0]
    onehot = (b[:, None] == lax.broadcasted_iota(b.dtype, (1, G), 1)).astype(jnp.float32)
    ps = lax.dot_general(onehot, h, (((0,), (0,)), ((), ())),
                         preferred_element_type=jnp.float32)
    cs = jnp.sum(onehot, axis=0)

    @pl.when(i == 0)
    def _():
        psum_ref[...] = ps
        cnt_ref[...] = cs

    @pl.when(i != 0)
    def _():
        psum_ref[...] = psum_ref[...] + ps
        cnt_ref[...] = cnt_ref[...] + cs

    @pl.when(i == NGRID - 1)
    def _():
        pooled = psum_ref[...] / jnp.maximum(cnt_ref[...], 1.0)[:, None]
        o = jnp.dot(pooled, w1_ref[...], preferred_element_type=jnp.float32)
        o = jnp.maximum(o + b1_ref[...], 0.0)
        o2 = jnp.dot(o, w2_ref[...], preferred_element_type=jnp.float32)
        out_ref[...] = o2[:, 0] + b2_ref[...]


def _row_spec(width):
    return pl.BlockSpec((BN, width), lambda i: (i, 0))


def _full_spec(shape):
    nd = len(shape)
    return pl.BlockSpec(shape, lambda i: (0,) * nd)


def _tc_layer(first, xin, hprev, bprev, wp, asel, bsel):
    body = _tc_first_body if first else _tc_mid_body
    win_w = DIN if first else HID
    out_shapes = [
        jax.ShapeDtypeStruct((N, HID), jnp.float32),
        jax.ShapeDtypeStruct((N, HEADS * HID), jnp.float32),
        jax.ShapeDtypeStruct((N, AB), jnp.float32),
        jax.ShapeDtypeStruct((HEADS,), jnp.float32),
        jax.ShapeDtypeStruct((HEADS,), jnp.float32),
    ]
    in_specs = [
        _row_spec(win_w if first else HID),
        _full_spec((win_w, HID)) if first else _row_spec(HID),
        _full_spec((HID,)),
        _full_spec((HID, HEADS * HID)),
        _full_spec((HEADS * HID, HEADS)),
        _full_spec((HEADS * HID, HEADS)),
    ]
    out_specs = [
        _row_spec(HID),
        _row_spec(HEADS * HID),
        _row_spec(AB),
        _full_spec((HEADS,)),
        _full_spec((HEADS,)),
    ]
    return pl.pallas_call(
        body,
        grid=(NGRID,),
        in_specs=in_specs,
        out_specs=out_specs,
        out_shape=out_shapes,
    )(xin, hprev, bprev, wp, asel, bsel)


def _tc_pool(acc, hprev, bprev, batch3, w1, b1, w2, b2):
    out_shapes = [
        jax.ShapeDtypeStruct((G,), jnp.float32),
        jax.ShapeDtypeStruct((G, HID), jnp.float32),
        jax.ShapeDtypeStruct((G,), jnp.float32),
    ]
    in_specs = [
        _row_spec(HID),
        _row_spec(HID),
        _full_spec((HID,)),
        pl.BlockSpec((1, 1, BN), lambda i: (i, 0, 0)),
        _full_spec((HID, HID)),
        _full_spec((HID,)),
        _full_spec((HID, 1)),
        _full_spec((1,)),
    ]
    out_specs = [
        _full_spec((G,)),
        _full_spec((G, HID)),
        _full_spec((G,)),
    ]
    res = pl.pallas_call(
        _tc_pool_body,
        grid=(NGRID,),
        in_specs=in_specs,
        out_specs=out_specs,
        out_shape=out_shapes,
    )(acc, hprev, bprev, batch3, w1, b1, w2, b2)
    return res[0]


def _sc_layer_body(src2, dst2, ab_h, hw8, c16, z8,
                   out_h,
                   src_a, dst_a, aev, src_b, dst_b, idx_b,
                   asr_a, adr_a, asr_b, adr_b, denr_b, rows_b, mv, cv,
                   den_sp, acc_sp):
    c = lax.axis_index("c")
    s = lax.axis_index("s")
    iota = lax.iota(jnp.int32, 16)

    r0 = s * ROWS_PT
    pltpu.sync_copy(z8, den_sp.at[pl.ds(r0, ROWS_PT)])
    pltpu.sync_copy(z8, acc_sp.at[pl.ds(r0, ROWS_PT)])
    pltpu.sync_copy(c16, cv)
    pltpu.sync_copy(z8.at[pl.ds(0, BLK_A)], aev)
    plsc.subcore_barrier()

    base_e = s * EPT

    chs = [cv[h] for h in range(HEADS)]

    def phase_a(b, carry):
        e0 = base_e + b * BLK_A
        for j in range(4):
            pltpu.sync_copy(src2.at[pl.ds(e0 + j * 128, 128)], src_a.at[j])
            pltpu.sync_copy(dst2.at[pl.ds(e0 + j * 128, 128)], dst_a.at[j])
        for j in range(4):
            pltpu.sync_copy(ab_h.at[src_a.at[j]], asr_a.at[pl.ds(j * 128, 128)])
            pltpu.sync_copy(ab_h.at[dst_a.at[j]], adr_a.at[pl.ds(j * 128, 128)])
        for h in range(HEADS):
            hvec = jnp.full((16,), h, jnp.int32)
            hvec4 = jnp.full((16,), HEADS + h, jnp.int32)

            def grp(g, _, h=h, hvec=hvec, hvec4=hvec4, e0=e0):
                evec = g * 16 + iota
                a = (plsc.load_gather(asr_a, [evec, hvec])
                     + plsc.load_gather(adr_a, [evec, hvec4]))
                a = jnp.where(a > 0, a, a * _NEG)
                ae = jnp.exp(a - chs[h])
                ae = jnp.where(e0 + evec < E_ALL, ae, 0.0)
                plsc.store_scatter(aev, [evec, hvec], ae)
                return 0

            lax.fori_loop(0, BLK_A // 16, grp, 0)
        for j in range(4):
            pltpu.sync_copy(aev.at[pl.ds(j * 128, 128)],
                            den_sp.at[dst_a.at[j]], add=True)
        return 0

    lax.fori_loop(0, NBLK_A, phase_a, 0)
    plsc.subcore_barrier()

    for p in range(NQ // 2):
        q = c * (NQ // 2) + p

        def phase_b(b, carry, q=q):
            e0 = base_e + b * BLK_B
            for j in range(2):
                pltpu.sync_copy(src2.at[pl.ds(e0 + j * 128, 128)], src_b.at[j])
                pltpu.sync_copy(dst2.at[pl.ds(e0 + j * 128, 128)], dst_b.at[j])
            for j in range(2):
                def mkidx(g, _, j=j):
                    sv = src_b[j, pl.ds(g * 16, 16)]
                    idx_b[j, pl.ds(g * 16, 16)] = sv * NQ + q
                    return 0

                lax.fori_loop(0, 8, mkidx, 0)
            for j in range(2):
                pltpu.sync_copy(hw8.at[idx_b.at[j]], rows_b.at[pl.ds(j * 128, 128)])
                pltpu.sync_copy(ab_h.at[src_b.at[j]], asr_b.at[pl.ds(j * 128, 128)])
                pltpu.sync_copy(ab_h.at[dst_b.at[j]], adr_b.at[pl.ds(j * 128, 128)])
                pltpu.sync_copy(den_sp.at[dst_b.at[j]], denr_b.at[pl.ds(j * 128, 128)])

            def grp(g, _, e0=e0):
                evec = g * 16 + iota
                inb = e0 + evec < E_ALL
                ws = []
                for h in range(HEADS):
                    hvec = jnp.full((16,), h, jnp.int32)
                    hvec4 = jnp.full((16,), HEADS + h, jnp.int32)
                    a = (plsc.load_gather(asr_b, [evec, hvec])
                         + plsc.load_gather(adr_b, [evec, hvec4]))
                    a = jnp.where(a > 0, a, a * _NEG)
                    ae = jnp.exp(a - chs[h])
                    den = plsc.load_gather(denr_b, [evec, hvec])
                    w = ae / (den + 1e-16)
                    ws.append(jnp.where(inb, w, 0.0))
                for d in range(QW):
                    acc = ws[0] * plsc.load_gather(
                        rows_b, [evec, jnp.full((16,), d, jnp.int32)])
                    for h in range(1, HEADS):
                        acc = acc + ws[h] * plsc.load_gather(
                            rows_b, [evec, jnp.full((16,), h * QW + d, jnp.int32)])
                    plsc.store_scatter(mv, [evec, jnp.full((16,), d, jnp.int32)], acc)
                return 0

            lax.fori_loop(0, BLK_B // 16, grp, 0)
            for j in range(2):
                pltpu.sync_copy(mv.at[pl.ds(j * 128, 128)],
                                acc_sp.at[dst_b.at[j]], add=True)
            return 0

        lax.fori_loop(0, NBLK_B, phase_b, 0)
        plsc.subcore_barrier()
        pltpu.sync_copy(acc_sp.at[pl.ds(r0, ROWS_PT)],
                        out_h.at[q, pl.ds(r0, ROWS_PT)])
        if p != NQ // 2 - 1:
            pltpu.sync_copy(z8, acc_sp.at[pl.ds(r0, ROWS_PT)])
            plsc.subcore_barrier()


def _sc_layer(src2, dst2, ab, hwp, c16, z8):
    mesh = plsc.VectorSubcoreMesh(core_axis_name="c", subcore_axis_name="s")
    hw8 = hwp.reshape(NQ * N, HEADS * QW)
    f = pl.kernel(
        _sc_layer_body,
        out_type=jax.ShapeDtypeStruct((NQ, N_PAD, QW), jnp.float32),
        mesh=mesh,
        scratch_types=[
            pltpu.VMEM((4, 128), jnp.int32),
            pltpu.VMEM((4, 128), jnp.int32),
            pltpu.VMEM((BLK_A, AB), jnp.float32),
            pltpu.VMEM((2, 128), jnp.int32),
            pltpu.VMEM((2, 128), jnp.int32),
            pltpu.VMEM((2, 128), jnp.int32),
            pltpu.VMEM((BLK_A, AB), jnp.float32),
            pltpu.VMEM((BLK_A, AB), jnp.float32),
            pltpu.VMEM((BLK_B, AB), jnp.float32),
            pltpu.VMEM((BLK_B, AB), jnp.float32),
            pltpu.VMEM((BLK_B, AB), jnp.float32),
            pltpu.VMEM((BLK_B, HEADS * QW), jnp.float32),
            pltpu.VMEM((BLK_B, QW), jnp.float32),
            pltpu.VMEM((HEADS, 16), jnp.float32),
            pltpu.VMEM_SHARED((N_PAD, AB), jnp.float32),
            pltpu.VMEM_SHARED((N_PAD, QW), jnp.float32),
        ],
        compiler_params=pltpu.CompilerParams(needs_layout_passes=False,
                                             use_tc_tiling_on_sc=False),
    )
    return f(src2, dst2, ab, hw8, c16, z8)


_kk, _hh, _jj = _np.meshgrid(_np.arange(NQ), _np.arange(HEADS), _np.arange(QW),
                             indexing="ij")
_PERM = (_hh * HID + _kk * QW + _jj).reshape(-1)
_HH_F = _hh.reshape(-1)
_DD_F = (_kk * QW + _jj).reshape(-1)


def _prep_weights(w, a_s, a_d):
    wp = w[:, _PERM]
    asel = jnp.zeros((HEADS * HID, HEADS), jnp.float32).at[
        _np.arange(HEADS * HID), _HH_F].set(a_s[_HH_F, _DD_F])
    bsel = jnp.zeros((HEADS * HID, HEADS), jnp.float32).at[
        _np.arange(HEADS * HID), _HH_F].set(a_d[_HH_F, _DD_F])
    return wp, asel, bsel


def kernel(x, edge_index, batch, W_in, b_in, W0, a_src0, a_dst0, b0,
           W1, a_src1, a_dst1, b1, W2, a_src2, a_dst2, b2,
           W_fc1, b_fc1, W_fc2, b_fc2):
    loop = jnp.arange(N, dtype=jnp.int32)
    src = jnp.concatenate([edge_index[0].astype(jnp.int32), loop])
    dst = jnp.concatenate([edge_index[1].astype(jnp.int32), loop])
    src2 = jnp.pad(src, (0, E_PAD - E_ALL))
    dst2 = jnp.pad(dst, (0, E_PAD - E_ALL))
    z8 = jnp.zeros((ROWS_PT, AB), jnp.float32)
    batch3 = batch.astype(jnp.int32).reshape(NGRID, 1, BN)

    layers = ((W0, a_src0, a_dst0, b0), (W1, a_src1, a_dst1, b1),
              (W2, a_src2, a_dst2, b2))

    hprev = x
    acc = None
    bias_prev = None
    for li, (w, a_s, a_d, bb) in enumerate(layers):
        wp, asel, bsel = _prep_weights(w, a_s, a_d)
        if li == 0:
            h, hwp, ab, ms, md = _tc_layer(True, x, W_in, b_in, wp, asel, bsel)
        else:
            h, hwp, ab, ms, md = _tc_layer(False, acc, hprev, bias_prev,
                                           wp, asel, bsel)
        cbound = jnp.maximum(ms + md, 0.0)
        c16 = jnp.broadcast_to(cbound[:, None], (HEADS, 16))
        out = _sc_layer(src2, dst2, ab, hwp, c16, z8)
        acc = out[:, :N].transpose(1, 0, 2).reshape(N, HID)
        hprev = h
        bias_prev = bb

    return _tc_pool(acc, hprev, bias_prev, batch3, W_fc1, b_fc1, W_fc2, b_fc2)


# speedup vs baseline: 1.9126x; 1.9126x over previous
import functools

import jax
import jax.numpy as jnp
import numpy as _np
from jax import lax
from jax.experimental import pallas as pl
from jax.experimental.pallas import tpu as pltpu
from jax.experimental.pallas import tpu_sc as plsc

N = 50000
DIN = 128
HID = 64
HEADS = 4
G = 64
NQ = 8
QW = HID // NQ
AB = 8

E_RAW = 800000
E_ALL = E_RAW + N
NSUB = 16
EPT = 53248
E_PAD = EPT * NSUB
BLK_A = 512
BLK_B = 256
NBLK_A = EPT // BLK_A
NBLK_B = EPT // BLK_B
N_PAD = 50048
ROWS_PT = N_PAD // NSUB
BN = 1000
NGRID = N // BN

_NEG = 0.2


def _tc_first_body(x_ref, win_ref, bin_ref, wp_ref, asel_ref, bsel_ref,
                   h_ref, hwp_ref, ab_ref, ms_ref, md_ref):
    i = pl.program_id(0)
    h = jnp.dot(x_ref[...], win_ref[...], preferred_element_type=jnp.float32)
    h = h + bin_ref[...]
    h_ref[...] = h
    hwp = jnp.dot(h, wp_ref[...], preferred_element_type=jnp.float32)
    hwp_ref[...] = hwp
    als = jnp.dot(hwp, asel_ref[...], preferred_element_type=jnp.float32)
    ald = jnp.dot(hwp, bsel_ref[...], preferred_element_type=jnp.float32)
    ab_ref[...] = jnp.concatenate([als, ald], axis=1)
    bs = jnp.max(als, axis=0)
    bd = jnp.max(ald, axis=0)

    @pl.when(i == 0)
    def _():
        ms_ref[...] = bs
        md_ref[...] = bd

    @pl.when(i != 0)
    def _():
        ms_ref[...] = jnp.maximum(ms_ref[...], bs)
        md_ref[...] = jnp.maximum(md_ref[...], bd)


def _tc_mid_body(acc_ref, hp_ref, bp_ref, wp_ref, asel_ref, bsel_ref,
                 h_ref, hwp_ref, ab_ref, ms_ref, md_ref):
    i = pl.program_id(0)
    t = acc_ref[...] * (1.0 / HEADS) + bp_ref[...]
    h = jnp.where(t > 0, t, jnp.exp(t) - 1.0) + hp_ref[...]
    h_ref[...] = h
    hwp = jnp.dot(h, wp_ref[...], preferred_element_type=jnp.float32)
    hwp_ref[...] = hwp
    als = jnp.dot(hwp, asel_ref[...], preferred_element_type=jnp.float32)
    ald = jnp.dot(hwp, bsel_ref[...], preferred_element_type=jnp.float32)
    ab_ref[...] = jnp.concatenate([als, ald], axis=1)
    bs = jnp.max(als, axis=0)
    bd = jnp.max(ald, axis=0)

    @pl.when(i == 0)
    def _():
        ms_ref[...] = bs
        md_ref[...] = bd

    @pl.when(i != 0)
    def _():
        ms_ref[...] = jnp.maximum(ms_ref[...], bs)
        md_ref[...] = jnp.maximum(md_ref[...], bd)


def _tc_pool_body(acc_ref, hp_ref, bp_ref, batch_ref, w1_ref, b1_ref,
                  w2_ref, b2_ref, out_ref, psum_ref, cnt_ref):
    i = pl.program_id(0)
    t = acc_ref[...] * (1.0 / HEADS) + bp_ref[...]
    h = jnp.where(t > 0, t, jnp.exp(t) - 1.0) + hp_ref[...]
    b = batch_ref[0, 0]
    onehot = (b[:, None] == lax.broadcasted_iota(b.dtype, (1, G), 1)).astype(jnp.float32)
    ps = lax.dot_general(onehot, h, (((0,), (0,)), ((), ())),
                         preferred_element_type=jnp.float32)
    cs = jnp.sum(onehot, axis=0)

    @pl.when(i == 0)
    def _():
        psum_ref[...] = ps
        cnt_ref[...] = cs

    @pl.when(i != 0)
    def _():
        psum_ref[...] = psum_ref[...] + ps
        cnt_ref[...] = cnt_ref[...] + cs

    @pl.when(i == NGRID - 1)
    def _():
        pooled = psum_ref[...] / jnp.maximum(cnt_ref[...], 1.0)[:, None]
        o = jnp.dot(pooled, w1_ref[...], preferred_element_type=jnp.float32)
        o = jnp.maximum(o + b1_ref[...], 0.0)
        o2 = jnp.dot(o, w2_ref[...], preferred_element_type=jnp.float32)
        out_ref[...] = o2[:, 0] + b2_ref[...]


def _row_spec(width):
    return pl.BlockSpec((BN, width), lambda i: (i, 0))


def _full_spec(shape):
    nd = len(shape)
    return pl.BlockSpec(shape, lambda i: (0,) * nd)


def _tc_layer(first, xin, hprev, bprev, wp, asel, bsel):
    body = _tc_first_body if first else _tc_mid_body
    win_w = DIN if first else HID
    out_shapes = [
        jax.ShapeDtypeStruct((N, HID), jnp.float32),
        jax.ShapeDtypeStruct((N, HEADS * HID), jnp.float32),
        jax.ShapeDtypeStruct((N, AB), jnp.float32),
        jax.ShapeDtypeStruct((HEADS,), jnp.float32),
        jax.ShapeDtypeStruct((HEADS,), jnp.float32),
    ]
    in_specs = [
        _row_spec(win_w if first else HID),
        _full_spec((win_w, HID)) if first else _row_spec(HID),
        _full_spec((HID,)),
        _full_spec((HID, HEADS * HID)),
        _full_spec((HEADS * HID, HEADS)),
        _full_spec((HEADS * HID, HEADS)),
    ]
    out_specs = [
        _row_spec(HID),
        _row_spec(HEADS * HID),
        _row_spec(AB),
        _full_spec((HEADS,)),
        _full_spec((HEADS,)),
    ]
    return pl.pallas_call(
        body,
        grid=(NGRID,),
        in_specs=in_specs,
        out_specs=out_specs,
        out_shape=out_shapes,
    )(xin, hprev, bprev, wp, asel, bsel)


def _tc_pool(acc, hprev, bprev, batch3, w1, b1, w2, b2):
    out_shapes = [
        jax.ShapeDtypeStruct((G,), jnp.float32),
        jax.ShapeDtypeStruct((G, HID), jnp.float32),
        jax.ShapeDtypeStruct((G,), jnp.float32),
    ]
    in_specs = [
        _row_spec(HID),
        _row_spec(HID),
        _full_spec((HID,)),
        pl.BlockSpec((1, 1, BN), lambda i: (i, 0, 0)),
        _full_spec((HID, HID)),
        _full_spec((HID,)),
        _full_spec((HID, 1)),
        _full_spec((1,)),
    ]
    out_specs = [
        _full_spec((G,)),
        _full_spec((G, HID)),
        _full_spec((G,)),
    ]
    res = pl.pallas_call(
        _tc_pool_body,
        grid=(NGRID,),
        in_specs=in_specs,
        out_specs=out_specs,
        out_shape=out_shapes,
    )(acc, hprev, bprev, batch3, w1, b1, w2, b2)
    return res[0]


def _sc_layer_body(src2, dst2, ab_h, hw8, c16, z8,
                   out_h, ae_out, w_out,
                   src_a, dst_a, aev, idx_b,
                   asr, adr, denr, wv, rows_b, mv, cv, sem,
                   den_sp, acc_sp):
    c = lax.axis_index("c")
    s = lax.axis_index("s")
    iota = lax.iota(jnp.int32, 16)
    ae_c = ae_out.at[c]
    w_c = w_out.at[c]

    r0 = s * ROWS_PT
    pltpu.sync_copy(z8, den_sp.at[pl.ds(r0, ROWS_PT)])
    pltpu.sync_copy(z8, acc_sp.at[pl.ds(r0, ROWS_PT)])
    pltpu.sync_copy(c16, cv)
    pltpu.sync_copy(z8.at[pl.ds(0, BLK_A)], aev)
    pltpu.sync_copy(z8.at[pl.ds(0, BLK_A)], wv)
    plsc.subcore_barrier()

    base_e = s * EPT

    chs = [cv[h] for h in range(HEADS)]

    def fire(copies):
        ds = []
        for k, cp in enumerate(copies):
            add = len(cp) == 3 and cp[2]
            ds.append(pltpu.async_copy(cp[0], cp[1], sem.at[k], add=add))
        for d in ds:
            d.wait()

    def phase_a(b, carry):
        e0 = base_e + b * BLK_A
        fire([(src2.at[pl.ds(e0 + j * 128, 128)], src_a.at[j]) for j in range(4)]
             + [(dst2.at[pl.ds(e0 + j * 128, 128)], dst_a.at[j]) for j in range(4)])
        fire([(ab_h.at[src_a.at[j]], asr.at[pl.ds(j * 128, 128)]) for j in range(4)]
             + [(ab_h.at[dst_a.at[j]], adr.at[pl.ds(j * 128, 128)]) for j in range(4)])
        for h in range(HEADS):
            hvec = jnp.full((16,), h, jnp.int32)
            hvec4 = jnp.full((16,), HEADS + h, jnp.int32)

            def grp(g, _, h=h, hvec=hvec, hvec4=hvec4, e0=e0):
                evec = g * 16 + iota
                a = (plsc.load_gather(asr, [evec, hvec])
                     + plsc.load_gather(adr, [evec, hvec4]))
                a = jnp.where(a > 0, a, a * _NEG)
                ae = jnp.exp(a - chs[h])
                ae = jnp.where(e0 + evec < E_ALL, ae, 0.0)
                plsc.store_scatter(aev, [evec, hvec], ae)
                return 0

            lax.fori_loop(0, BLK_A // 16, grp, 0)
        fire([(aev.at[pl.ds(j * 128, 128)], den_sp.at[dst_a.at[j]], True)
              for j in range(4)]
             + [(aev, ae_c.at[pl.ds(e0, BLK_A)])])
        return 0

    lax.fori_loop(0, NBLK_A, phase_a, 0)
    plsc.subcore_barrier()

    def phase_a2(b, carry):
        e0 = base_e + b * BLK_A
        fire([(dst2.at[pl.ds(e0 + j * 128, 128)], dst_a.at[j]) for j in range(4)])
        fire([(den_sp.at[dst_a.at[j]], denr.at[pl.ds(j * 128, 128)])
              for j in range(4)]
             + [(ae_c.at[pl.ds(e0, BLK_A)], aev)])
        for h in range(HEADS):
            hvec = jnp.full((16,), h, jnp.int32)

            def grp(g, _, hvec=hvec):
                evec = g * 16 + iota
                ae = plsc.load_gather(aev, [evec, hvec])
                den = plsc.load_gather(denr, [evec, hvec])
                plsc.store_scatter(wv, [evec, hvec], ae / (den + 1e-16))
                return 0

            lax.fori_loop(0, BLK_A // 16, grp, 0)
        fire([(wv, w_c.at[pl.ds(e0, BLK_A)])])
        return 0

    lax.fori_loop(0, NBLK_A, phase_a2, 0)

    for p in range(NQ // 2):
        q = c * (NQ // 2) + p

        def phase_b(b, carry, q=q):
            e0 = base_e + b * BLK_A
            fire([(src2.at[pl.ds(e0 + j * 128, 128)], src_a.at[j])
                  for j in range(4)]
                 + [(dst2.at[pl.ds(e0 + j * 128, 128)], dst_a.at[j])
                    for j in range(4)])
            for j in range(4):
                def mkidx(g, _, j=j):
                    sv = src_a[j, pl.ds(g * 16, 16)]
                    idx_b[j, pl.ds(g * 16, 16)] = sv * NQ + q
                    return 0

                lax.fori_loop(0, 8, mkidx, 0)
            fire([(hw8.at[idx_b.at[j]], rows_b.at[pl.ds(j * 128, 128)])
                  for j in range(4)]
                 + [(w_c.at[pl.ds(e0, BLK_A)], wv)])

            def grp(g, _):
                evec = g * 16 + iota
                ws = [plsc.load_gather(wv, [evec, jnp.full((16,), h, jnp.int32)])
                      for h in range(HEADS)]
                for d in range(QW):
                    acc = ws[0] * plsc.load_gather(
                        rows_b, [evec, jnp.full((16,), d, jnp.int32)])
                    for h in range(1, HEADS):
                        acc = acc + ws[h] * plsc.load_gather(
                            rows_b, [evec, jnp.full((16,), h * QW + d, jnp.int32)])
                    plsc.store_scatter(mv, [evec, jnp.full((16,), d, jnp.int32)], acc)
                return 0

            lax.fori_loop(0, BLK_A // 16, grp, 0)
            fire([(mv.at[pl.ds(j * 128, 128)], acc_sp.at[dst_a.at[j]], True)
                  for j in range(4)])
            return 0

        lax.fori_loop(0, NBLK_A, phase_b, 0)
        plsc.subcore_barrier()
        pltpu.sync_copy(acc_sp.at[pl.ds(r0, ROWS_PT)],
                        out_h.at[q, pl.ds(r0, ROWS_PT)])
        if p != NQ // 2 - 1:
            pltpu.sync_copy(z8, acc_sp.at[pl.ds(r0, ROWS_PT)])
            plsc.subcore_barrier()


def _sc_layer(src2, dst2, ab, hwp, c16, z8):
    mesh = plsc.VectorSubcoreMesh(core_axis_name="c", subcore_axis_name="s")
    hw8 = hwp.reshape(NQ * N, HEADS * QW)
    f = pl.kernel(
        _sc_layer_body,
        out_type=[
            jax.ShapeDtypeStruct((NQ, N_PAD, QW), jnp.float32),
            jax.ShapeDtypeStruct((2, E_PAD, AB), jnp.float32),
            jax.ShapeDtypeStruct((2, E_PAD, AB), jnp.float32),
        ],
        mesh=mesh,
        scratch_types=[
            pltpu.VMEM((4, 128), jnp.int32),
            pltpu.VMEM((4, 128), jnp.int32),
            pltpu.VMEM((BLK_A, AB), jnp.float32),
            pltpu.VMEM((4, 128), jnp.int32),
            pltpu.VMEM((BLK_A, AB), jnp.float32),
            pltpu.VMEM((BLK_A, AB), jnp.float32),
            pltpu.VMEM((BLK_A, AB), jnp.float32),
            pltpu.VMEM((BLK_A, AB), jnp.float32),
            pltpu.VMEM((BLK_A, HEADS * QW), jnp.float32),
            pltpu.VMEM((BLK_A, QW), jnp.float32),
            pltpu.VMEM((HEADS, 16), jnp.float32),
            pltpu.SemaphoreType.DMA((9,)),
            pltpu.VMEM_SHARED((N_PAD, AB), jnp.float32),
            pltpu.VMEM_SHARED((N_PAD, QW), jnp.float32),
        ],
        compiler_params=pltpu.CompilerParams(needs_layout_passes=False,
                                             use_tc_tiling_on_sc=False),
    )
    out, _, _ = f(src2, dst2, ab, hw8, c16, z8)
    return out


_kk, _hh, _jj = _np.meshgrid(_np.arange(NQ), _np.arange(HEADS), _np.arange(QW),
                             indexing="ij")
_PERM = (_hh * HID + _kk * QW + _jj).reshape(-1)
_HH_F = _hh.reshape(-1)
_DD_F = (_kk * QW + _jj).reshape(-1)


def _prep_weights(w, a_s, a_d):
    wp = w[:, _PERM]
    asel = jnp.zeros((HEADS * HID, HEADS), jnp.float32).at[
        _np.arange(HEADS * HID), _HH_F].set(a_s[_HH_F, _DD_F])
    bsel = jnp.zeros((HEADS * HID, HEADS), jnp.float32).at[
        _np.arange(HEADS * HID), _HH_F].set(a_d[_HH_F, _DD_F])
    return wp, asel, bsel


def kernel(x, edge_index, batch, W_in, b_in, W0, a_src0, a_dst0, b0,
           W1, a_src1, a_dst1, b1, W2, a_src2, a_dst2, b2,
           W_fc1, b_fc1, W_fc2, b_fc2):
    loop = jnp.arange(N, dtype=jnp.int32)
    src = jnp.concatenate([edge_index[0].astype(jnp.int32), loop])
    dst = jnp.concatenate([edge_index[1].astype(jnp.int32), loop])
    src2 = jnp.pad(src, (0, E_PAD - E_ALL))
    dst2 = jnp.pad(dst, (0, E_PAD - E_ALL))
    z8 = jnp.zeros((ROWS_PT, AB), jnp.float32)
    batch3 = batch.astype(jnp.int32).reshape(NGRID, 1, BN)

    layers = ((W0, a_src0, a_dst0, b0), (W1, a_src1, a_dst1, b1),
              (W2, a_src2, a_dst2, b2))

    hprev = x
    acc = None
    bias_prev = None
    for li, (w, a_s, a_d, bb) in enumerate(layers):
        wp, asel, bsel = _prep_weights(w, a_s, a_d)
        if li == 0:
            h, hwp, ab, ms, md = _tc_layer(True, x, W_in, b_in, wp, asel, bsel)
        else:
            h, hwp, ab, ms, md = _tc_layer(False, acc, hprev, bias_prev,
                                           wp, asel, bsel)
        cbound = jnp.maximum(ms + md, 0.0)
        c16 = jnp.broadcast_to(cbound[:, None], (HEADS, 16))
        out = _sc_layer(src2, dst2, ab, hwp, c16, z8)
        acc = out[:, :N].transpose(1, 0, 2).reshape(N, HID)
        hprev = h
        bias_prev = bb

    return _tc_pool(acc, hprev, bias_prev, batch3, W_fc1, b_fc1, W_fc2, b_fc2)


# speedup vs baseline: 2.0085x; 1.0502x over previous
import functools

import jax
import jax.numpy as jnp
import numpy as _np
from jax import lax
from jax.experimental import pallas as pl
from jax.experimental.pallas import tpu as pltpu
from jax.experimental.pallas import tpu_sc as plsc

N = 50000
DIN = 128
HID = 64
HEADS = 4
G = 64
NQ = 8
QW = HID // NQ
AB = 8

E_RAW = 800000
E_ALL = E_RAW + N
NSUB = 16
EPT = 53248
E_PAD = EPT * NSUB
BLK_A = 1024
NCH = BLK_A // 128
NBLK_A = EPT // BLK_A
N_PAD = 50048
ROWS_PT = N_PAD // NSUB
BN = 1000
NGRID = N // BN

_NEG = 0.2


def _tc_first_body(x_ref, win_ref, bin_ref, wp_ref, asel_ref, bsel_ref,
                   h_ref, hwp_ref, ab_ref, ms_ref, md_ref):
    i = pl.program_id(0)
    h = jnp.dot(x_ref[...], win_ref[...], preferred_element_type=jnp.float32)
    h = h + bin_ref[...]
    h_ref[...] = h
    hwp = jnp.dot(h, wp_ref[...], preferred_element_type=jnp.float32)
    hwp_ref[...] = hwp
    als = jnp.dot(hwp, asel_ref[...], preferred_element_type=jnp.float32)
    ald = jnp.dot(hwp, bsel_ref[...], preferred_element_type=jnp.float32)
    ab_ref[...] = jnp.concatenate([als, ald], axis=1)
    bs = jnp.max(als, axis=0)
    bd = jnp.max(ald, axis=0)

    @pl.when(i == 0)
    def _():
        ms_ref[...] = bs
        md_ref[...] = bd

    @pl.when(i != 0)
    def _():
        ms_ref[...] = jnp.maximum(ms_ref[...], bs)
        md_ref[...] = jnp.maximum(md_ref[...], bd)


def _tc_mid_body(acc_ref, hp_ref, bp_ref, wp_ref, asel_ref, bsel_ref,
                 h_ref, hwp_ref, ab_ref, ms_ref, md_ref):
    i = pl.program_id(0)
    t = acc_ref[...] * (1.0 / HEADS) + bp_ref[...]
    h = jnp.where(t > 0, t, jnp.exp(t) - 1.0) + hp_ref[...]
    h_ref[...] = h
    hwp = jnp.dot(h, wp_ref[...], preferred_element_type=jnp.float32)
    hwp_ref[...] = hwp
    als = jnp.dot(hwp, asel_ref[...], preferred_element_type=jnp.float32)
    ald = jnp.dot(hwp, bsel_ref[...], preferred_element_type=jnp.float32)
    ab_ref[...] = jnp.concatenate([als, ald], axis=1)
    bs = jnp.max(als, axis=0)
    bd = jnp.max(ald, axis=0)

    @pl.when(i == 0)
    def _():
        ms_ref[...] = bs
        md_ref[...] = bd

    @pl.when(i != 0)
    def _():
        ms_ref[...] = jnp.maximum(ms_ref[...], bs)
        md_ref[...] = jnp.maximum(md_ref[...], bd)


def _tc_pool_body(acc_ref, hp_ref, bp_ref, batch_ref, w1_ref, b1_ref,
                  w2_ref, b2_ref, out_ref, psum_ref, cnt_ref):
    i = pl.program_id(0)
    t = acc_ref[...] * (1.0 / HEADS) + bp_ref[...]
    h = jnp.where(t > 0, t, jnp.exp(t) - 1.0) + hp_ref[...]
    b = batch_ref[0, 0]
    onehot = (b[:, None] == lax.broadcasted_iota(b.dtype, (1, G), 1)).astype(jnp.float32)
    ps = lax.dot_general(onehot, h, (((0,), (0,)), ((), ())),
                         preferred_element_type=jnp.float32)
    cs = jnp.sum(onehot, axis=0)

    @pl.when(i == 0)
    def _():
        psum_ref[...] = ps
        cnt_ref[...] = cs

    @pl.when(i != 0)
    def _():
        psum_ref[...] = psum_ref[...] + ps
        cnt_ref[...] = cnt_ref[...] + cs

    @pl.when(i == NGRID - 1)
    def _():
        pooled = psum_ref[...] / jnp.maximum(cnt_ref[...], 1.0)[:, None]
        o = jnp.dot(pooled, w1_ref[...], preferred_element_type=jnp.float32)
        o = jnp.maximum(o + b1_ref[...], 0.0)
        o2 = jnp.dot(o, w2_ref[...], preferred_element_type=jnp.float32)
        out_ref[...] = o2[:, 0] + b2_ref[...]


def _row_spec(width):
    return pl.BlockSpec((BN, width), lambda i: (i, 0))


def _full_spec(shape):
    nd = len(shape)
    return pl.BlockSpec(shape, lambda i: (0,) * nd)


def _tc_layer(first, xin, hprev, bprev, wp, asel, bsel):
    body = _tc_first_body if first else _tc_mid_body
    win_w = DIN if first else HID
    out_shapes = [
        jax.ShapeDtypeStruct((N, HID), jnp.float32),
        jax.ShapeDtypeStruct((N, HEADS * HID), jnp.float32),
        jax.ShapeDtypeStruct((N, AB), jnp.float32),
        jax.ShapeDtypeStruct((HEADS,), jnp.float32),
        jax.ShapeDtypeStruct((HEADS,), jnp.float32),
    ]
    in_specs = [
        _row_spec(win_w if first else HID),
        _full_spec((win_w, HID)) if first else _row_spec(HID),
        _full_spec((HID,)),
        _full_spec((HID, HEADS * HID)),
        _full_spec((HEADS * HID, HEADS)),
        _full_spec((HEADS * HID, HEADS)),
    ]
    out_specs = [
        _row_spec(HID),
        _row_spec(HEADS * HID),
        _row_spec(AB),
        _full_spec((HEADS,)),
        _full_spec((HEADS,)),
    ]
    return pl.pallas_call(
        body,
        grid=(NGRID,),
        in_specs=in_specs,
        out_specs=out_specs,
        out_shape=out_shapes,
    )(xin, hprev, bprev, wp, asel, bsel)


def _tc_pool(acc, hprev, bprev, batch3, w1, b1, w2, b2):
    out_shapes = [
        jax.ShapeDtypeStruct((G,), jnp.float32),
        jax.ShapeDtypeStruct((G, HID), jnp.float32),
        jax.ShapeDtypeStruct((G,), jnp.float32),
    ]
    in_specs = [
        _row_spec(HID),
        _row_spec(HID),
        _full_spec((HID,)),
        pl.BlockSpec((1, 1, BN), lambda i: (i, 0, 0)),
        _full_spec((HID, HID)),
        _full_spec((HID,)),
        _full_spec((HID, 1)),
        _full_spec((1,)),
    ]
    out_specs = [
        _full_spec((G,)),
        _full_spec((G, HID)),
        _full_spec((G,)),
    ]
    res = pl.pallas_call(
        _tc_pool_body,
        grid=(NGRID,),
        in_specs=in_specs,
        out_specs=out_specs,
        out_shape=out_shapes,
    )(acc, hprev, bprev, batch3, w1, b1, w2, b2)
    return res[0]


def _sc_layer_body(src2, dst2, ab_h, hw8, c16, z8,
                   out_h, ae_out, w_out,
                   src_a, dst_a, aev, idx_b,
                   asr, adr, denr, wv, rows_b, mv, cv, sem,
                   nd_sp):
    c = lax.axis_index("c")
    s = lax.axis_index("s")
    iota = lax.iota(jnp.int32, 16)
    ae_c = ae_out.at[c]
    w_c = w_out.at[c]

    r0 = s * ROWS_PT
    pltpu.sync_copy(z8, nd_sp.at[pl.ds(r0, ROWS_PT)])
    pltpu.sync_copy(c16, cv)
    pltpu.sync_copy(z8.at[pl.ds(0, BLK_A)], aev)
    pltpu.sync_copy(z8.at[pl.ds(0, BLK_A)], wv)
    plsc.subcore_barrier()

    base_e = s * EPT

    chs = [cv[h] for h in range(HEADS)]

    def cp(a, b, k, add=False):
        return pltpu.async_copy(a, b, sem.at[k], add=add)

    def phase_a(b, carry):
        e0 = base_e + b * BLK_A
        ds = ([cp(src2.at[pl.ds(e0 + j * 128, 128)], src_a.at[j], j)
               for j in range(NCH)]
              + [cp(dst2.at[pl.ds(e0 + j * 128, 128)], dst_a.at[j], NCH + j)
                 for j in range(NCH)])
        for d in ds:
            d.wait()
        gs = [cp(ab_h.at[src_a.at[j]], asr.at[pl.ds(j * 128, 128)], j)
              for j in range(NCH)]
        gd = [cp(ab_h.at[dst_a.at[j]], adr.at[pl.ds(j * 128, 128)], NCH + j)
              for j in range(NCH)]
        tail = []
        for j in range(NCH):
            gs[j].wait()
            gd[j].wait()
            for h in range(HEADS):
                hvec = jnp.full((16,), h, jnp.int32)
                hvec4 = jnp.full((16,), HEADS + h, jnp.int32)

                def grp(g, _, h=h, hvec=hvec, hvec4=hvec4, e0=e0, j=j):
                    evec = j * 128 + g * 16 + iota
                    a = (plsc.load_gather(asr, [evec, hvec])
                         + plsc.load_gather(adr, [evec, hvec4]))
                    a = jnp.where(a > 0, a, a * _NEG)
                    ae = jnp.exp(a - chs[h])
                    ae = jnp.where(e0 + evec < E_ALL, ae, 0.0)
                    plsc.store_scatter(aev, [evec, hvec], ae)
                    return 0

                lax.fori_loop(0, 8, grp, 0)
            tail.append(cp(aev.at[pl.ds(j * 128, 128)],
                           nd_sp.at[dst_a.at[j]], j, add=True))
        tail.append(cp(aev, ae_c.at[pl.ds(e0, BLK_A)], 2 * NCH))
        for d in tail:
            d.wait()
        return 0

    lax.fori_loop(0, NBLK_A, phase_a, 0)
    plsc.subcore_barrier()

    def phase_a2(b, carry):
        e0 = base_e + b * BLK_A
        ds = [cp(dst2.at[pl.ds(e0 + j * 128, 128)], dst_a.at[j], j)
              for j in range(NCH)]
        for d in ds:
            d.wait()
        ld = cp(ae_c.at[pl.ds(e0, BLK_A)], aev, 2 * NCH)
        gd = [cp(nd_sp.at[dst_a.at[j]], denr.at[pl.ds(j * 128, 128)], j)
              for j in range(NCH)]
        ld.wait()
        tail = []
        for j in range(NCH):
            gd[j].wait()
            for h in range(HEADS):
                hvec = jnp.full((16,), h, jnp.int32)

                def grp(g, _, hvec=hvec, j=j):
                    evec = j * 128 + g * 16 + iota
                    ae = plsc.load_gather(aev, [evec, hvec])
                    den = plsc.load_gather(denr, [evec, hvec])
                    plsc.store_scatter(wv, [evec, hvec], ae / (den + 1e-16))
                    return 0

                lax.fori_loop(0, 8, grp, 0)
        tail.append(cp(wv, w_c.at[pl.ds(e0, BLK_A)], 2 * NCH))
        for d in tail:
            d.wait()
        return 0

    lax.fori_loop(0, NBLK_A, phase_a2, 0)
    plsc.subcore_barrier()
    pltpu.sync_copy(z8, nd_sp.at[pl.ds(r0, ROWS_PT)])
    plsc.subcore_barrier()

    def pass_body(p, pcarry):
        q = c * (NQ // 2) + p

        def phase_b(b, carry, q=q):
            e0 = base_e + b * BLK_A
            ds = ([cp(src2.at[pl.ds(e0 + j * 128, 128)], src_a.at[j], j)
                   for j in range(NCH)]
                  + [cp(dst2.at[pl.ds(e0 + j * 128, 128)], dst_a.at[j], NCH + j)
                     for j in range(NCH)])
            for d in ds:
                d.wait()
            for j in range(NCH):
                def mkidx(g, _, j=j):
                    sv = src_a[j, pl.ds(g * 16, 16)]
                    idx_b[j, pl.ds(g * 16, 16)] = sv * NQ + q
                    return 0

                lax.fori_loop(0, 8, mkidx, 0)
            ld = cp(w_c.at[pl.ds(e0, BLK_A)], wv, 2 * NCH)
            gh = [cp(hw8.at[idx_b.at[j]], rows_b.at[pl.ds(j * 128, 128)], j)
                  for j in range(NCH)]
            ld.wait()
            tail = []
            for j in range(NCH):
                gh[j].wait()

                def grp(g, _, j=j):
                    evec = j * 128 + g * 16 + iota
                    ws = [plsc.load_gather(wv, [evec, jnp.full((16,), h, jnp.int32)])
                          for h in range(HEADS)]
                    for d in range(QW):
                        acc = ws[0] * plsc.load_gather(
                            rows_b, [evec, jnp.full((16,), d, jnp.int32)])
                        for h in range(1, HEADS):
                            acc = acc + ws[h] * plsc.load_gather(
                                rows_b, [evec, jnp.full((16,), h * QW + d, jnp.int32)])
                        plsc.store_scatter(mv, [evec, jnp.full((16,), d, jnp.int32)],
                                           acc)
                    return 0

                lax.fori_loop(0, 8, grp, 0)
                tail.append(cp(mv.at[pl.ds(j * 128, 128)],
                               nd_sp.at[dst_a.at[j]], j, add=True))
            for d in tail:
                d.wait()
            return 0

        lax.fori_loop(0, NBLK_A, phase_b, 0)
        plsc.subcore_barrier()
        pltpu.sync_copy(nd_sp.at[pl.ds(r0, ROWS_PT)],
                        out_h.at[q, pl.ds(r0, ROWS_PT)])
        pltpu.sync_copy(z8, nd_sp.at[pl.ds(r0, ROWS_PT)])
        plsc.subcore_barrier()
        return 0

    lax.fori_loop(0, NQ // 2, pass_body, 0)


def _sc_layer(src2, dst2, ab, hwp, c16, z8):
    mesh = plsc.VectorSubcoreMesh(core_axis_name="c", subcore_axis_name="s")
    hw8 = hwp.reshape(NQ * N, HEADS * QW)
    f = pl.kernel(
        _sc_layer_body,
        out_type=[
            jax.ShapeDtypeStruct((NQ, N_PAD, QW), jnp.float32),
            jax.ShapeDtypeStruct((2, E_PAD, AB), jnp.float32),
            jax.ShapeDtypeStruct((2, E_PAD, AB), jnp.float32),
        ],
        mesh=mesh,
        scratch_types=[
            pltpu.VMEM((NCH, 128), jnp.int32),
            pltpu.VMEM((NCH, 128), jnp.int32),
            pltpu.VMEM((BLK_A, AB), jnp.float32),
            pltpu.VMEM((NCH, 128), jnp.int32),
            pltpu.VMEM((BLK_A, AB), jnp.float32),
            pltpu.VMEM((BLK_A, AB), jnp.float32),
            pltpu.VMEM((BLK_A, AB), jnp.float32),
            pltpu.VMEM((BLK_A, AB), jnp.float32),
            pltpu.VMEM((BLK_A, HEADS * QW), jnp.float32),
            pltpu.VMEM((BLK_A, QW), jnp.float32),
            pltpu.VMEM((HEADS, 16), jnp.float32),
            pltpu.SemaphoreType.DMA((2 * NCH + 1,)),
            pltpu.VMEM_SHARED((N_PAD, AB), jnp.float32),
        ],
        compiler_params=pltpu.CompilerParams(needs_layout_passes=False,
                                             use_tc_tiling_on_sc=False),
    )
    out, _, _ = f(src2, dst2, ab, hw8, c16, z8)
    return out


_kk, _hh, _jj = _np.meshgrid(_np.arange(NQ), _np.arange(HEADS), _np.arange(QW),
                             indexing="ij")
_PERM = (_hh * HID + _kk * QW + _jj).reshape(-1)
_HH_F = _hh.reshape(-1)
_DD_F = (_kk * QW + _jj).reshape(-1)


def _prep_weights(w, a_s, a_d):
    wp = w[:, _PERM]
    asel = jnp.zeros((HEADS * HID, HEADS), jnp.float32).at[
        _np.arange(HEADS * HID), _HH_F].set(a_s[_HH_F, _DD_F])
    bsel = jnp.zeros((HEADS * HID, HEADS), jnp.float32).at[
        _np.arange(HEADS * HID), _HH_F].set(a_d[_HH_F, _DD_F])
    return wp, asel, bsel


def kernel(x, edge_index, batch, W_in, b_in, W0, a_src0, a_dst0, b0,
           W1, a_src1, a_dst1, b1, W2, a_src2, a_dst2, b2,
           W_fc1, b_fc1, W_fc2, b_fc2):
    loop = jnp.arange(N, dtype=jnp.int32)
    src = jnp.concatenate([edge_index[0].astype(jnp.int32), loop])
    dst = jnp.concatenate([edge_index[1].astype(jnp.int32), loop])
    src2 = jnp.pad(src, (0, E_PAD - E_ALL))
    dst2 = jnp.pad(dst, (0, E_PAD - E_ALL))
    z8 = jnp.zeros((ROWS_PT, AB), jnp.float32)
    batch3 = batch.astype(jnp.int32).reshape(NGRID, 1, BN)

    layers = ((W0, a_src0, a_dst0, b0), (W1, a_src1, a_dst1, b1),
              (W2, a_src2, a_dst2, b2))

    hprev = x
    acc = None
    bias_prev = None
    for li, (w, a_s, a_d, bb) in enumerate(layers):
        wp, asel, bsel = _prep_weights(w, a_s, a_d)
        if li == 0:
            h, hwp, ab, ms, md = _tc_layer(True, x, W_in, b_in, wp, asel, bsel)
        else:
            h, hwp, ab, ms, md = _tc_layer(False, acc, hprev, bias_prev,
                                           wp, asel, bsel)
        cbound = jnp.maximum(ms + md, 0.0)
        c16 = jnp.broadcast_to(cbound[:, None], (HEADS, 16))
        out = _sc_layer(src2, dst2, ab, hwp, c16, z8)
        acc = out[:, :N].transpose(1, 0, 2).reshape(N, HID)
        hprev = h
        bias_prev = bb

    return _tc_pool(acc, hprev, bias_prev, batch3, W_fc1, b_fc1, W_fc2, b_fc2)
